# Initial kernel scaffold; baseline (speedup 1.0000x reference)
import functools

import jax
import jax.numpy as jnp
from jax import lax
from jax.experimental import pallas as pl
from jax.experimental.pallas import tpu as pltpu
from jax.experimental.pallas import tpu_sc as plsc

_L = 16
_CH = 128


def _wh_body(x_ref, w_ref, b_ref, a2_ref, wh_ref, s12_ref):
    wh = jnp.dot(x_ref[...], w_ref[...], preferred_element_type=jnp.float32)
    wh = wh + b_ref[...]
    wh_ref[...] = wh
    s12_ref[...] = jnp.dot(wh, a2_ref[...], preferred_element_type=jnp.float32)


def _fin_body(wh_ref, a0_ref, a1_ref, d0_ref, d1_ref, o_ref):
    d = d0_ref[0] + d1_ref[0]
    agg = a0_ref[0] + a1_ref[0]
    safe = jnp.where(d > 0, d, 1.0)
    o_ref[...] = jnp.where(d > 0, agg / safe, wh_ref[...])


def _sc_edge_kernel(n_nodes, n_edges, out_dim,
                    edge_hbm, s12_hbm, wh_hbm, zrow_hbm, zden_hbm,
                    agg_out, den_out,
                    s12_v, src_v, dst_v, w_v, rows_v, agg_sh, den_sh, sem):
    c = lax.axis_index("c")
    s = lax.axis_index("s")
    nc = 2
    nw = 32
    tile = s * nc + c

    pltpu.sync_copy(s12_hbm, s12_v)

    @pl.when(s == 0)
    def _():
        pltpu.sync_copy(zrow_hbm, agg_sh)
        pltpu.sync_copy(zden_hbm, den_sh)

    plsc.subcore_barrier()

    n_chunks = n_edges // _CH
    n_mine = (n_chunks - 1 - tile) // nw + 1

    zeros16 = jnp.zeros((_L,), jnp.int32)
    ones16 = jnp.ones((_L,), jnp.int32)

    def chunk_body(i, carry):
        base = (tile + i * nw) * _CH
        pltpu.sync_copy(edge_hbm.at[0, pl.ds(base, _CH)], src_v)
        pltpu.sync_copy(edge_hbm.at[1, pl.ds(base, _CH)], dst_v)
        gather = pltpu.async_copy(wh_hbm.at[dst_v], rows_v, sem)
        for q in range(_CH // _L):
            sl = pl.ds(q * _L, _L)
            si = src_v[sl]
            di = dst_v[sl]
            s1 = plsc.load_gather(s12_v, [si, zeros16])
            s2 = plsc.load_gather(s12_v, [di, ones16])
            v = s1 + s2
            e = jnp.where(v > 0, v, 0.2 * v)
            w_v[sl] = jnp.exp(e)
        gather.wait()

        def scale_body(j, carry2):
            wj = w_v[j]
            for q in range(out_dim // _L):
                sl = pl.ds(q * _L, _L)
                rows_v[j, sl] = rows_v[j, sl] * wj
            return carry2

        lax.fori_loop(0, _CH, scale_body, 0)

        pltpu.sync_copy(rows_v, agg_sh.at[src_v], add=True)
        pltpu.sync_copy(w_v, den_sh.at[src_v], add=True)
        return carry

    lax.fori_loop(0, n_mine, chunk_body, 0)

    plsc.subcore_barrier()

    @pl.when(s == 0)
    def _():
        pltpu.sync_copy(agg_sh, agg_out.at[c])
        pltpu.sync_copy(den_sh, den_out.at[c])


def kernel(x, edge_index, W_w, W_b, a, We_w, We_b):
    n, in_dim = x.shape
    out_dim = W_w.shape[0]
    n_edges = edge_index.shape[1]

    wt = W_w.T
    bias = W_b.reshape(1, out_dim)
    a2 = jnp.stack([a[0, :out_dim], a[0, out_dim:]], axis=1)

    bn = 1000
    grid = n // bn

    wh, s12 = pl.pallas_call(
        _wh_body,
        grid=(grid,),
        in_specs=[
            pl.BlockSpec((bn, in_dim), lambda i: (i, 0)),
            pl.BlockSpec((in_dim, out_dim), lambda i: (0, 0)),
            pl.BlockSpec((1, out_dim), lambda i: (0, 0)),
            pl.BlockSpec((out_dim, 2), lambda i: (0, 0)),
        ],
        out_specs=[
            pl.BlockSpec((bn, out_dim), lambda i: (i, 0)),
            pl.BlockSpec((bn, 2), lambda i: (i, 0)),
        ],
        out_shape=[
            jax.ShapeDtypeStruct((n, out_dim), jnp.float32),
            jax.ShapeDtypeStruct((n, 2), jnp.float32),
        ],
    )(x, wt, bias, a2)

    zrow = jnp.zeros((n, out_dim), jnp.float32)
    zden = jnp.zeros((n,), jnp.float32)

    mesh = plsc.VectorSubcoreMesh(core_axis_name="c", subcore_axis_name="s")
    sc_fn = pl.kernel(
        functools.partial(_sc_edge_kernel, n, n_edges, out_dim),
        out_type=[
            jax.ShapeDtypeStruct((2, n, out_dim), jnp.float32),
            jax.ShapeDtypeStruct((2, n), jnp.float32),
        ],
        mesh=mesh,
        scratch_types=[
            pltpu.VMEM((n, 2), jnp.float32),
            pltpu.VMEM((_CH,), jnp.int32),
            pltpu.VMEM((_CH,), jnp.int32),
            pltpu.VMEM((_CH,), jnp.float32),
            pltpu.VMEM((_CH, out_dim), jnp.float32),
            pltpu.VMEM_SHARED((n, out_dim), jnp.float32),
            pltpu.VMEM_SHARED((n,), jnp.float32),
            pltpu.SemaphoreType.DMA,
        ],
    )
    agg_part, den_part = sc_fn(edge_index, s12, wh, zrow, zden)

    den3 = den_part.reshape(2, n, 1)
    out = pl.pallas_call(
        _fin_body,
        grid=(grid,),
        in_specs=[
            pl.BlockSpec((bn, out_dim), lambda i: (i, 0)),
            pl.BlockSpec((1, bn, out_dim), lambda i: (0, i, 0)),
            pl.BlockSpec((1, bn, out_dim), lambda i: (1, i, 0)),
            pl.BlockSpec((1, bn, 1), lambda i: (0, i, 0)),
            pl.BlockSpec((1, bn, 1), lambda i: (1, i, 0)),
        ],
        out_specs=pl.BlockSpec((bn, out_dim), lambda i: (i, 0)),
        out_shape=jax.ShapeDtypeStruct((n, out_dim), jnp.float32),
    )(wh, agg_part, agg_part, den3, den3)

    return out


# speedup vs baseline: 44.2603x; 44.2603x over previous
import functools

import jax
import jax.numpy as jnp
from jax import lax
from jax.experimental import pallas as pl
from jax.experimental.pallas import tpu as pltpu
from jax.experimental.pallas import tpu_sc as plsc

_L = 16
_CH = 128


def _wh_body(x_ref, w_ref, b_ref, a2_ref, wh_ref, s12_ref):
    wh = jnp.dot(x_ref[...], w_ref[...], preferred_element_type=jnp.float32)
    wh = wh + b_ref[...]
    wh_ref[...] = wh
    s12_ref[...] = jnp.dot(wh, a2_ref[...], preferred_element_type=jnp.float32)


def _fin_body(wh_ref, a0_ref, a1_ref, d0_ref, d1_ref, o_ref):
    d = d0_ref[0] + d1_ref[0]
    agg = a0_ref[0] + a1_ref[0]
    safe = jnp.where(d > 0, d, 1.0)
    o_ref[...] = jnp.where(d > 0, agg / safe, wh_ref[...])


def _sc_edge_kernel(n_nodes, n_edges, out_dim,
                    edge_hbm, s12t_hbm, wh_hbm, zrow_hbm, zden_hbm,
                    agg_out, den_out,
                    s1_v, s2_v, src_v, dst_v, w_v, rows_v, agg_sh, den_sh,
                    sem):
    c = lax.axis_index("c")
    s = lax.axis_index("s")
    nc = 2
    nw = 32
    tile = s * nc + c

    pltpu.sync_copy(s12t_hbm.at[0], s1_v)
    pltpu.sync_copy(s12t_hbm.at[1], s2_v)

    @pl.when(s == 0)
    def _():
        pltpu.sync_copy(zrow_hbm, agg_sh)
        pltpu.sync_copy(zden_hbm, den_sh)

    plsc.subcore_barrier()

    n_chunks = n_edges // _CH
    n_mine = (n_chunks - 1 - tile) // nw + 1

    def chunk_body(i, carry):
        base = (tile + i * nw) * _CH
        pltpu.sync_copy(edge_hbm.at[0, pl.ds(base, _CH)], src_v)
        pltpu.sync_copy(edge_hbm.at[1, pl.ds(base, _CH)], dst_v)
        gather = pltpu.async_copy(wh_hbm.at[dst_v], rows_v, sem)
        for q in range(_CH // _L):
            sl = pl.ds(q * _L, _L)
            si = src_v[sl]
            di = dst_v[sl]
            s1 = plsc.load_gather(s1_v, [si])
            s2 = plsc.load_gather(s2_v, [di])
            v = s1 + s2
            e = jnp.where(v > 0, v, 0.2 * v)
            w_v[sl] = jnp.exp(e)
        gather.wait()

        def scale_body(g, carry2):
            base2 = g * _L
            wv = w_v[pl.ds(base2, _L)]
            for j in range(_L):
                wj = wv[j]
                for q in range(out_dim // _L):
                    sl = pl.ds(q * _L, _L)
                    rows_v[base2 + j, sl] = rows_v[base2 + j, sl] * wj
            return carry2

        lax.fori_loop(0, _CH // _L, scale_body, 0)

        pltpu.sync_copy(rows_v, agg_sh.at[src_v], add=True)
        pltpu.sync_copy(w_v, den_sh.at[src_v], add=True)
        return carry

    lax.fori_loop(0, n_mine, chunk_body, 0)

    plsc.subcore_barrier()

    @pl.when(s == 0)
    def _():
        pltpu.sync_copy(agg_sh, agg_out.at[c])
        pltpu.sync_copy(den_sh, den_out.at[c])


def kernel(x, edge_index, W_w, W_b, a, We_w, We_b):
    n, in_dim = x.shape
    out_dim = W_w.shape[0]
    n_edges = edge_index.shape[1]

    wt = W_w.T
    bias = W_b.reshape(1, out_dim)
    a2 = jnp.stack([a[0, :out_dim], a[0, out_dim:]], axis=1)

    bn = 1000
    grid = n // bn

    wh, s12 = pl.pallas_call(
        _wh_body,
        grid=(grid,),
        in_specs=[
            pl.BlockSpec((bn, in_dim), lambda i: (i, 0)),
            pl.BlockSpec((in_dim, out_dim), lambda i: (0, 0)),
            pl.BlockSpec((1, out_dim), lambda i: (0, 0)),
            pl.BlockSpec((out_dim, 2), lambda i: (0, 0)),
        ],
        out_specs=[
            pl.BlockSpec((bn, out_dim), lambda i: (i, 0)),
            pl.BlockSpec((bn, 2), lambda i: (i, 0)),
        ],
        out_shape=[
            jax.ShapeDtypeStruct((n, out_dim), jnp.float32),
            jax.ShapeDtypeStruct((n, 2), jnp.float32),
        ],
    )(x, wt, bias, a2)

    zrow = jnp.zeros((n, out_dim), jnp.float32)
    zden = jnp.zeros((n,), jnp.float32)

    mesh = plsc.VectorSubcoreMesh(core_axis_name="c", subcore_axis_name="s")
    sc_fn = pl.kernel(
        functools.partial(_sc_edge_kernel, n, n_edges, out_dim),
        out_type=[
            jax.ShapeDtypeStruct((2, n, out_dim), jnp.float32),
            jax.ShapeDtypeStruct((2, n), jnp.float32),
        ],
        mesh=mesh,
        compiler_params=pltpu.CompilerParams(needs_layout_passes=False),
        scratch_types=[
            pltpu.VMEM((n,), jnp.float32),
            pltpu.VMEM((n,), jnp.float32),
            pltpu.VMEM((_CH,), jnp.int32),
            pltpu.VMEM((_CH,), jnp.int32),
            pltpu.VMEM((_CH,), jnp.float32),
            pltpu.VMEM((_CH, out_dim), jnp.float32),
            pltpu.VMEM_SHARED((n, out_dim), jnp.float32),
            pltpu.VMEM_SHARED((n,), jnp.float32),
            pltpu.SemaphoreType.DMA,
        ],
    )
    agg_part, den_part = sc_fn(edge_index, s12.T, wh, zrow, zden)

    den3 = den_part.reshape(2, n, 1)
    out = pl.pallas_call(
        _fin_body,
        grid=(grid,),
        in_specs=[
            pl.BlockSpec((bn, out_dim), lambda i: (i, 0)),
            pl.BlockSpec((1, bn, out_dim), lambda i: (0, i, 0)),
            pl.BlockSpec((1, bn, out_dim), lambda i: (1, i, 0)),
            pl.BlockSpec((1, bn, 1), lambda i: (0, i, 0)),
            pl.BlockSpec((1, bn, 1), lambda i: (1, i, 0)),
        ],
        out_specs=pl.BlockSpec((bn, out_dim), lambda i: (i, 0)),
        out_shape=jax.ShapeDtypeStruct((n, out_dim), jnp.float32),
    )(wh, agg_part, agg_part, den3, den3)

    return out


# speedup vs baseline: 54.9525x; 1.2416x over previous
import functools

import jax
import jax.numpy as jnp
from jax import lax
from jax.experimental import pallas as pl
from jax.experimental.pallas import tpu as pltpu
from jax.experimental.pallas import tpu_sc as plsc

_L = 16
_CH = 128


def _wh_body(x_ref, w_ref, b_ref, a2_ref, wh_ref, s12_ref):
    wh = jnp.dot(x_ref[...], w_ref[...], preferred_element_type=jnp.float32)
    wh = wh + b_ref[...]
    wh_ref[...] = wh
    s12_ref[...] = jnp.dot(wh, a2_ref[...], preferred_element_type=jnp.float32)


def _fin_body(wh_ref, a0_ref, a1_ref, d0_ref, d1_ref, o_ref):
    d = d0_ref[0] + d1_ref[0]
    agg = a0_ref[0] + a1_ref[0]
    safe = jnp.where(d > 0, d, 1.0)
    o_ref[...] = jnp.where(d > 0, agg / safe, wh_ref[...])


def _sc_edge_kernel(n_nodes, n_edges, out_dim,
                    edge_hbm, s1_hbm, s2_hbm, wh_hbm, zrow_hbm, zden_hbm,
                    agg_out, den_out,
                    s1c_v, s2c_v, src_v, dst_v, w_v, rows_v, agg_sh, den_sh,
                    sem_i0, sem_i1, sem_g0, sem_g1, sem_s0, sem_s1):
    c = lax.axis_index("c")
    s = lax.axis_index("s")
    nc = 2
    nw = 32
    tile = s * nc + c
    sem_i = (sem_i0, sem_i1)
    sem_g = (sem_g0, sem_g1)
    sem_s = (sem_s0, sem_s1)

    @pl.when(s == 0)
    def _():
        pltpu.sync_copy(zrow_hbm, agg_sh)
        pltpu.sync_copy(zden_hbm, den_sh)

    plsc.subcore_barrier()

    n_chunks = n_edges // _CH
    n_mine = (n_chunks - 1 - tile) // nw + 1

    def issue_idx(ci, b):
        base = (tile + ci * nw) * _CH
        pltpu.async_copy(edge_hbm.at[0, pl.ds(base, _CH)], src_v.at[b],
                         sem_i[b])
        pltpu.async_copy(edge_hbm.at[1, pl.ds(base, _CH)], dst_v.at[b],
                         sem_i[b])

    def wait_scatter(b):
        pltpu.make_async_copy(rows_v.at[b], agg_sh.at[src_v.at[b]],
                              sem_s[b]).wait()
        pltpu.make_async_copy(w_v.at[b], den_sh.at[src_v.at[b]],
                              sem_s[b]).wait()

    def chunk_step(ci, b):
        ob = 1 - b
        sv = src_v.at[b]
        dv = dst_v.at[b]
        wv_ref = w_v.at[b]
        rv = rows_v.at[b]
        s1c = s1c_v.at[b]
        s2c = s2c_v.at[b]
        pltpu.make_async_copy(edge_hbm.at[0, pl.ds(0, _CH)], sv,
                              sem_i[b]).wait()
        pltpu.make_async_copy(edge_hbm.at[1, pl.ds(0, _CH)], dv,
                              sem_i[b]).wait()
        pltpu.async_copy(wh_hbm.at[dv], rv, sem_g[b])
        pltpu.async_copy(s1_hbm.at[sv], s1c, sem_g[b])
        pltpu.async_copy(s2_hbm.at[dv], s2c, sem_g[b])
        pltpu.make_async_copy(wh_hbm.at[dv], rv, sem_g[b]).wait()
        pltpu.make_async_copy(s1_hbm.at[sv], s1c, sem_g[b]).wait()
        pltpu.make_async_copy(s2_hbm.at[dv], s2c, sem_g[b]).wait()
        for q in range(_CH // _L):
            sl = pl.ds(q * _L, _L)
            v = s1c[sl] + s2c[sl]
            e = jnp.where(v > 0, v, 0.2 * v)
            wv_ref[sl] = jnp.exp(e)

        def scale_body(g, carry2):
            base2 = g * _L
            wv = wv_ref[pl.ds(base2, _L)]
            for j in range(_L):
                wj = wv[j]
                for q in range(out_dim // _L):
                    sl = pl.ds(q * _L, _L)
                    rv[base2 + j, sl] = rv[base2 + j, sl] * wj
            return carry2

        lax.fori_loop(0, _CH // _L, scale_body, 0)

        @pl.when(ci >= 1)
        def _():
            wait_scatter(ob)

        @pl.when(ci + 1 < n_mine)
        def _():
            issue_idx(ci + 1, ob)

        pltpu.async_copy(rv, agg_sh.at[sv], sem_s[b], add=True)
        pltpu.async_copy(wv_ref, den_sh.at[sv], sem_s[b], add=True)

    issue_idx(0, 0)

    def pair_body(p, carry):
        chunk_step(2 * p, 0)

        @pl.when(2 * p + 1 < n_mine)
        def _():
            chunk_step(2 * p + 1, 1)

        return carry

    lax.fori_loop(0, (n_mine + 1) // 2, pair_body, 0)

    last_parity = (n_mine - 1) % 2

    @pl.when(last_parity == 0)
    def _():
        wait_scatter(0)

    @pl.when(last_parity == 1)
    def _():
        wait_scatter(1)

    plsc.subcore_barrier()

    @pl.when(s == 0)
    def _():
        pltpu.sync_copy(agg_sh, agg_out.at[c])
        pltpu.sync_copy(den_sh, den_out.at[c])


def kernel(x, edge_index, W_w, W_b, a, We_w, We_b):
    n, in_dim = x.shape
    out_dim = W_w.shape[0]
    n_edges = edge_index.shape[1]

    wt = W_w.T
    bias = W_b.reshape(1, out_dim)
    a2 = jnp.stack([a[0, :out_dim], a[0, out_dim:]], axis=1)

    bn = 1000
    grid = n // bn

    wh, s12 = pl.pallas_call(
        _wh_body,
        grid=(grid,),
        in_specs=[
            pl.BlockSpec((bn, in_dim), lambda i: (i, 0)),
            pl.BlockSpec((in_dim, out_dim), lambda i: (0, 0)),
            pl.BlockSpec((1, out_dim), lambda i: (0, 0)),
            pl.BlockSpec((out_dim, 2), lambda i: (0, 0)),
        ],
        out_specs=[
            pl.BlockSpec((bn, out_dim), lambda i: (i, 0)),
            pl.BlockSpec((bn, 2), lambda i: (i, 0)),
        ],
        out_shape=[
            jax.ShapeDtypeStruct((n, out_dim), jnp.float32),
            jax.ShapeDtypeStruct((n, 2), jnp.float32),
        ],
    )(x, wt, bias, a2)

    zrow = jnp.zeros((n, out_dim), jnp.float32)
    zden = jnp.zeros((n,), jnp.float32)

    mesh = plsc.VectorSubcoreMesh(core_axis_name="c", subcore_axis_name="s")
    sc_fn = pl.kernel(
        functools.partial(_sc_edge_kernel, n, n_edges, out_dim),
        out_type=[
            jax.ShapeDtypeStruct((2, n, out_dim), jnp.float32),
            jax.ShapeDtypeStruct((2, n), jnp.float32),
        ],
        mesh=mesh,
        compiler_params=pltpu.CompilerParams(needs_layout_passes=False),
        scratch_types=[
            pltpu.VMEM((2, _CH), jnp.float32),
            pltpu.VMEM((2, _CH), jnp.float32),
            pltpu.VMEM((2, _CH), jnp.int32),
            pltpu.VMEM((2, _CH), jnp.int32),
            pltpu.VMEM((2, _CH), jnp.float32),
            pltpu.VMEM((2, _CH, out_dim), jnp.float32),
            pltpu.VMEM_SHARED((n, out_dim), jnp.float32),
            pltpu.VMEM_SHARED((n,), jnp.float32),
            pltpu.SemaphoreType.DMA,
            pltpu.SemaphoreType.DMA,
            pltpu.SemaphoreType.DMA,
            pltpu.SemaphoreType.DMA,
            pltpu.SemaphoreType.DMA,
            pltpu.SemaphoreType.DMA,
        ],
    )
    agg_part, den_part = sc_fn(edge_index, s12[:, 0], s12[:, 1], wh,
                               zrow, zden)

    den3 = den_part.reshape(2, n, 1)
    out = pl.pallas_call(
        _fin_body,
        grid=(grid,),
        in_specs=[
            pl.BlockSpec((bn, out_dim), lambda i: (i, 0)),
            pl.BlockSpec((1, bn, out_dim), lambda i: (0, i, 0)),
            pl.BlockSpec((1, bn, out_dim), lambda i: (1, i, 0)),
            pl.BlockSpec((1, bn, 1), lambda i: (0, i, 0)),
            pl.BlockSpec((1, bn, 1), lambda i: (1, i, 0)),
        ],
        out_specs=pl.BlockSpec((bn, out_dim), lambda i: (i, 0)),
        out_shape=jax.ShapeDtypeStruct((n, out_dim), jnp.float32),
    )(wh, agg_part, agg_part, den3, den3)

    return out


# speedup vs baseline: 61.3863x; 1.1171x over previous
import functools

import jax
import jax.numpy as jnp
from jax import lax
from jax.experimental import pallas as pl
from jax.experimental.pallas import tpu as pltpu
from jax.experimental.pallas import tpu_sc as plsc

_L = 16
_CH = 128


def _wh_body(x_ref, w_ref, b_ref, a2_ref, wh_ref, s12_ref):
    wh = jnp.dot(x_ref[...], w_ref[...], preferred_element_type=jnp.float32)
    wh = wh + b_ref[...]
    wh_ref[...] = wh
    s12_ref[...] = jnp.dot(wh, a2_ref[...], preferred_element_type=jnp.float32)


def _fin_body(wh_ref, a0_ref, a1_ref, d0_ref, d1_ref, o_ref):
    d = d0_ref[0] + d1_ref[0]
    agg = a0_ref[0] + a1_ref[0]
    safe = jnp.where(d > 0, d, 1.0)
    o_ref[...] = jnp.where(d > 0, agg / safe, wh_ref[...])


def _sc_edge_kernel(n_nodes, n_edges, out_dim,
                    edge_hbm, s1_hbm, s2_hbm, wh_hbm, zrow_hbm, zden_hbm,
                    agg_out, den_out,
                    s1c_v, s2c_v, src_v, dst_v, sidx_v, w_v, rows_v,
                    agg_sh, den_sh,
                    sem_i0, sem_i1, sem_g0, sem_g1, sem_s0, sem_s1):
    c = lax.axis_index("c")
    s = lax.axis_index("s")
    nc = 2
    nw = 32
    tile = s * nc + c
    sem_i = (sem_i0, sem_i1)
    sem_g = (sem_g0, sem_g1)
    sem_s = (sem_s0, sem_s1)

    @pl.when(s == 0)
    def _():
        pltpu.sync_copy(zrow_hbm, agg_sh)
        pltpu.sync_copy(zden_hbm, den_sh)

    plsc.subcore_barrier()

    n_chunks = n_edges // _CH
    n_mine = (n_chunks - 1 - tile) // nw + 1

    def issue_idx(ci, b):
        base = (tile + ci * nw) * _CH
        pltpu.async_copy(edge_hbm.at[0, pl.ds(base, _CH)], src_v.at[b],
                         sem_i[b])
        pltpu.async_copy(edge_hbm.at[1, pl.ds(base, _CH)], dst_v.at[b],
                         sem_i[b])

    def wait_scatter(b):
        pltpu.make_async_copy(rows_v.at[b], agg_sh.at[sidx_v.at[b]],
                              sem_s[b]).wait()
        pltpu.make_async_copy(w_v.at[b], den_sh.at[sidx_v.at[b]],
                              sem_s[b]).wait()

    def issue_gather(b):
        pltpu.async_copy(wh_hbm.at[dst_v.at[b]], rows_v.at[b], sem_g[b])
        pltpu.async_copy(s1_hbm.at[src_v.at[b]], s1c_v.at[b], sem_g[b])
        pltpu.async_copy(s2_hbm.at[dst_v.at[b]], s2c_v.at[b], sem_g[b])

    def wait_gather(b):
        pltpu.make_async_copy(wh_hbm.at[dst_v.at[b]], rows_v.at[b],
                              sem_g[b]).wait()
        pltpu.make_async_copy(s1_hbm.at[src_v.at[b]], s1c_v.at[b],
                              sem_g[b]).wait()
        pltpu.make_async_copy(s2_hbm.at[dst_v.at[b]], s2c_v.at[b],
                              sem_g[b]).wait()

    def chunk_step(ci, b):
        ob = 1 - b
        wv_ref = w_v.at[b]
        rv = rows_v.at[b]
        s1c = s1c_v.at[b]
        s2c = s2c_v.at[b]
        si = sidx_v.at[b]
        wait_gather(b)
        for q in range(_CH // _L):
            sl = pl.ds(q * _L, _L)
            v = s1c[sl] + s2c[sl]
            e = jnp.where(v > 0, v, 0.2 * v)
            wv_ref[sl] = jnp.exp(e)
        for q in range(_CH // _L):
            sl = pl.ds(q * _L, _L)
            si[sl] = src_v.at[b][sl]

        def scale_body(g, carry2):
            base2 = g * _L
            wv = wv_ref[pl.ds(base2, _L)]
            for j in range(_L):
                wj = wv[j]
                for q in range(out_dim // _L):
                    sl = pl.ds(q * _L, _L)
                    rv[base2 + j, sl] = rv[base2 + j, sl] * wj
            return carry2

        lax.fori_loop(0, _CH // _L, scale_body, 0)

        @pl.when(ci >= 1)
        def _():
            wait_scatter(ob)

        @pl.when(ci + 1 < n_mine)
        def _():
            pltpu.make_async_copy(edge_hbm.at[0, pl.ds(0, _CH)],
                                  src_v.at[ob], sem_i[ob]).wait()
            pltpu.make_async_copy(edge_hbm.at[1, pl.ds(0, _CH)],
                                  dst_v.at[ob], sem_i[ob]).wait()

        @pl.when(ci + 2 < n_mine)
        def _():
            issue_idx(ci + 2, b)

        @pl.when(ci + 1 < n_mine)
        def _():
            issue_gather(ob)

        pltpu.async_copy(rv, agg_sh.at[si], sem_s[b], add=True)
        pltpu.async_copy(wv_ref, den_sh.at[si], sem_s[b], add=True)

    issue_idx(0, 0)
    pltpu.make_async_copy(edge_hbm.at[0, pl.ds(0, _CH)], src_v.at[0],
                          sem_i[0]).wait()
    pltpu.make_async_copy(edge_hbm.at[1, pl.ds(0, _CH)], dst_v.at[0],
                          sem_i[0]).wait()
    issue_gather(0)

    @pl.when(1 < n_mine)
    def _():
        issue_idx(1, 1)

    def pair_body(p, carry):
        chunk_step(2 * p, 0)

        @pl.when(2 * p + 1 < n_mine)
        def _():
            chunk_step(2 * p + 1, 1)

        return carry

    lax.fori_loop(0, (n_mine + 1) // 2, pair_body, 0)

    last_parity = (n_mine - 1) % 2

    @pl.when(last_parity == 0)
    def _():
        wait_scatter(0)

    @pl.when(last_parity == 1)
    def _():
        wait_scatter(1)

    plsc.subcore_barrier()

    @pl.when(s == 0)
    def _():
        pltpu.sync_copy(agg_sh, agg_out.at[c])
        pltpu.sync_copy(den_sh, den_out.at[c])


def kernel(x, edge_index, W_w, W_b, a, We_w, We_b):
    n, in_dim = x.shape
    out_dim = W_w.shape[0]
    n_edges = edge_index.shape[1]

    wt = W_w.T
    bias = W_b.reshape(1, out_dim)
    a2 = jnp.stack([a[0, :out_dim], a[0, out_dim:]], axis=1)

    bn = 1000
    grid = n // bn

    wh, s12 = pl.pallas_call(
        _wh_body,
        grid=(grid,),
        in_specs=[
            pl.BlockSpec((bn, in_dim), lambda i: (i, 0)),
            pl.BlockSpec((in_dim, out_dim), lambda i: (0, 0)),
            pl.BlockSpec((1, out_dim), lambda i: (0, 0)),
            pl.BlockSpec((out_dim, 2), lambda i: (0, 0)),
        ],
        out_specs=[
            pl.BlockSpec((bn, out_dim), lambda i: (i, 0)),
            pl.BlockSpec((bn, 2), lambda i: (i, 0)),
        ],
        out_shape=[
            jax.ShapeDtypeStruct((n, out_dim), jnp.float32),
            jax.ShapeDtypeStruct((n, 2), jnp.float32),
        ],
    )(x, wt, bias, a2)

    zrow = jnp.zeros((n, out_dim), jnp.float32)
    zden = jnp.zeros((n,), jnp.float32)

    mesh = plsc.VectorSubcoreMesh(core_axis_name="c", subcore_axis_name="s")
    sc_fn = pl.kernel(
        functools.partial(_sc_edge_kernel, n, n_edges, out_dim),
        out_type=[
            jax.ShapeDtypeStruct((2, n, out_dim), jnp.float32),
            jax.ShapeDtypeStruct((2, n), jnp.float32),
        ],
        mesh=mesh,
        compiler_params=pltpu.CompilerParams(needs_layout_passes=False),
        scratch_types=[
            pltpu.VMEM((2, _CH), jnp.float32),
            pltpu.VMEM((2, _CH), jnp.float32),
            pltpu.VMEM((2, _CH), jnp.int32),
            pltpu.VMEM((2, _CH), jnp.int32),
            pltpu.VMEM((2, _CH), jnp.int32),
            pltpu.VMEM((2, _CH), jnp.float32),
            pltpu.VMEM((2, _CH, out_dim), jnp.float32),
            pltpu.VMEM_SHARED((n, out_dim), jnp.float32),
            pltpu.VMEM_SHARED((n,), jnp.float32),
            pltpu.SemaphoreType.DMA,
            pltpu.SemaphoreType.DMA,
            pltpu.SemaphoreType.DMA,
            pltpu.SemaphoreType.DMA,
            pltpu.SemaphoreType.DMA,
            pltpu.SemaphoreType.DMA,
        ],
    )
    agg_part, den_part = sc_fn(edge_index, s12[:, 0], s12[:, 1], wh,
                               zrow, zden)

    den3 = den_part.reshape(2, n, 1)
    out = pl.pallas_call(
        _fin_body,
        grid=(grid,),
        in_specs=[
            pl.BlockSpec((bn, out_dim), lambda i: (i, 0)),
            pl.BlockSpec((1, bn, out_dim), lambda i: (0, i, 0)),
            pl.BlockSpec((1, bn, out_dim), lambda i: (1, i, 0)),
            pl.BlockSpec((1, bn, 1), lambda i: (0, i, 0)),
            pl.BlockSpec((1, bn, 1), lambda i: (1, i, 0)),
        ],
        out_specs=pl.BlockSpec((bn, out_dim), lambda i: (i, 0)),
        out_shape=jax.ShapeDtypeStruct((n, out_dim), jnp.float32),
    )(wh, agg_part, agg_part, den3, den3)

    return out


# speedup vs baseline: 71.1497x; 1.1590x over previous
import functools

import jax
import jax.numpy as jnp
from jax import lax
from jax.experimental import pallas as pl
from jax.experimental.pallas import tpu as pltpu
from jax.experimental.pallas import tpu_sc as plsc

_L = 16
_CH = 112


def _wh_body(x_ref, w_ref, b_ref, a2_ref, wh_ref, s12_ref):
    wh = jnp.dot(x_ref[...], w_ref[...], preferred_element_type=jnp.float32)
    wh = wh + b_ref[...]
    wh_ref[...] = wh
    s12_ref[...] = jnp.dot(wh, a2_ref[...], preferred_element_type=jnp.float32)


def _fin_body(wh_ref, a0_ref, a1_ref, d0_ref, d1_ref, o_ref):
    d = d0_ref[0] + d1_ref[0]
    agg = a0_ref[0] + a1_ref[0]
    safe = jnp.where(d > 0, d, 1.0)
    o_ref[...] = jnp.where(d > 0, agg / safe, wh_ref[...])


def _sc_edge_kernel(n_nodes, n_edges, out_dim,
                    edge_hbm, s1_hbm, s2_hbm, wh_hbm, zrow_hbm, zden_hbm,
                    agg_out, den_out,
                    s1c_v, s2c_v, src_v, dst_v, sidx_v, w_v, rows_v,
                    tsrc_v, tdst_v, ts1_v, ts2_v, tw_v, trow_v,
                    agg_sh, den_sh,
                    sem_i0, sem_i1, sem_i2, sem_g0, sem_g1, sem_g2,
                    sem_s0, sem_s1, sem_s2, sem_t):
    c = lax.axis_index("c")
    s = lax.axis_index("s")
    nc = 2
    nw = 32
    tile = s * nc + c
    sem_i = (sem_i0, sem_i1, sem_i2)
    sem_g = (sem_g0, sem_g1, sem_g2)
    sem_s = (sem_s0, sem_s1, sem_s2)

    @pl.when(s == 0)
    def _():
        pltpu.sync_copy(zrow_hbm, agg_sh)
        pltpu.sync_copy(zden_hbm, den_sh)

    plsc.subcore_barrier()

    n_chunks = n_edges // _CH
    tail = n_edges - n_chunks * _CH
    n_mine = (n_chunks - 1 - tile) // nw + 1

    def issue_idx(ci, t):
        base = (tile + ci * nw) * _CH
        pltpu.async_copy(edge_hbm.at[pl.ds(base, _CH)], src_v.at[t],
                         sem_i[t])
        pltpu.async_copy(edge_hbm.at[pl.ds(n_edges + base, _CH)],
                         dst_v.at[t], sem_i[t])

    def wait_idx(t):
        pltpu.make_async_copy(edge_hbm.at[pl.ds(0, _CH)], src_v.at[t],
                              sem_i[t]).wait()
        pltpu.make_async_copy(edge_hbm.at[pl.ds(0, _CH)], dst_v.at[t],
                              sem_i[t]).wait()

    def issue_gather(t):
        pltpu.async_copy(wh_hbm.at[dst_v.at[t]], rows_v.at[t], sem_g[t])
        pltpu.async_copy(s1_hbm.at[src_v.at[t]], s1c_v.at[t], sem_g[t])
        pltpu.async_copy(s2_hbm.at[dst_v.at[t]], s2c_v.at[t], sem_g[t])

    def wait_gather(t):
        pltpu.make_async_copy(wh_hbm.at[dst_v.at[t]], rows_v.at[t],
                              sem_g[t]).wait()
        pltpu.make_async_copy(s1_hbm.at[src_v.at[t]], s1c_v.at[t],
                              sem_g[t]).wait()
        pltpu.make_async_copy(s2_hbm.at[dst_v.at[t]], s2c_v.at[t],
                              sem_g[t]).wait()

    def issue_scatter(t):
        pltpu.async_copy(rows_v.at[t], agg_sh.at[sidx_v.at[t]], sem_s[t],
                         add=True)
        pltpu.async_copy(w_v.at[t], den_sh.at[sidx_v.at[t]], sem_s[t],
                         add=True)

    def wait_scatter(t):
        pltpu.make_async_copy(rows_v.at[t], agg_sh.at[sidx_v.at[t]],
                              sem_s[t]).wait()
        pltpu.make_async_copy(w_v.at[t], den_sh.at[sidx_v.at[t]],
                              sem_s[t]).wait()

    def chunk_step(ci, t, tn, tp):
        wv_ref = w_v.at[t]
        rv = rows_v.at[t]
        s1c = s1c_v.at[t]
        s2c = s2c_v.at[t]
        si = sidx_v.at[t]
        sv = src_v.at[t]
        wait_gather(t)
        for q in range(_CH // _L):
            sl = pl.ds(q * _L, _L)
            v = s1c[sl] + s2c[sl]
            e = jnp.where(v > 0, v, 0.2 * v)
            wv_ref[sl] = jnp.exp(e)
            si[sl] = sv[sl]
        @pl.when(ci >= 2)
        def _():
            wait_scatter(tn)

        @pl.when(ci + 1 < n_mine)
        def _():
            wait_idx(tn)
            issue_gather(tn)

        def scale_body(g, carry2):
            base2 = g * _L
            wv = wv_ref[pl.ds(base2, _L)]
            for j in range(_L):
                wj = wv[j]
                for q in range(out_dim // _L):
                    sl = pl.ds(q * _L, _L)
                    rv[base2 + j, sl] = rv[base2 + j, sl] * wj
            return carry2

        lax.fori_loop(0, _CH // _L, scale_body, 0)

        @pl.when(ci + 2 < n_mine)
        def _():
            issue_idx(ci + 2, tp)

        issue_scatter(t)

    issue_idx(0, 0)
    wait_idx(0)
    issue_gather(0)

    @pl.when(1 < n_mine)
    def _():
        issue_idx(1, 1)

    def triple_body(p, carry):
        ci0 = 3 * p
        chunk_step(ci0, 0, 1, 2)

        @pl.when(ci0 + 1 < n_mine)
        def _():
            chunk_step(ci0 + 1, 1, 2, 0)

        @pl.when(ci0 + 2 < n_mine)
        def _():
            chunk_step(ci0 + 2, 2, 0, 1)

        return carry

    lax.fori_loop(0, (n_mine + 2) // 3, triple_body, 0)

    for k in range(3):
        @pl.when((n_mine - 2) % 3 == k)
        def _(k=k):
            wait_scatter(k)

    for k in range(3):
        @pl.when((n_mine - 1) % 3 == k)
        def _(k=k):
            wait_scatter(k)

    if tail:
        @pl.when(tile == 0)
        def _():
            base = n_chunks * _CH
            pltpu.sync_copy(edge_hbm.at[pl.ds(base, tail)], tsrc_v)
            pltpu.sync_copy(edge_hbm.at[pl.ds(n_edges + base, tail)],
                            tdst_v)
            trv = trow_v.at[0]
            pltpu.async_copy(wh_hbm.at[tdst_v], trv, sem_t)
            pltpu.async_copy(s1_hbm.at[tsrc_v], ts1_v, sem_t)
            pltpu.async_copy(s2_hbm.at[tdst_v], ts2_v, sem_t)
            pltpu.make_async_copy(wh_hbm.at[tdst_v], trv, sem_t).wait()
            pltpu.make_async_copy(s1_hbm.at[tsrc_v], ts1_v, sem_t).wait()
            pltpu.make_async_copy(s2_hbm.at[tdst_v], ts2_v, sem_t).wait()
            v = ts1_v[...] + ts2_v[...]
            e = jnp.where(v > 0, v, 0.2 * v)
            tw_v[...] = jnp.exp(e)
            wv = tw_v[...]
            for j in range(tail):
                wj = wv[j]
                for q in range(out_dim // _L):
                    sl = pl.ds(q * _L, _L)
                    trv[j, sl] = trv[j, sl] * wj
            pltpu.async_copy(trv, agg_sh.at[tsrc_v], sem_t, add=True)
            pltpu.async_copy(tw_v, den_sh.at[tsrc_v], sem_t, add=True)
            pltpu.make_async_copy(trv, agg_sh.at[tsrc_v], sem_t).wait()
            pltpu.make_async_copy(tw_v, den_sh.at[tsrc_v], sem_t).wait()

    plsc.subcore_barrier()

    @pl.when(s == 0)
    def _():
        pltpu.sync_copy(agg_sh, agg_out.at[c])
        pltpu.sync_copy(den_sh, den_out.at[c])


def kernel(x, edge_index, W_w, W_b, a, We_w, We_b):
    n, in_dim = x.shape
    out_dim = W_w.shape[0]
    n_edges = edge_index.shape[1]

    wt = W_w.T
    bias = W_b.reshape(1, out_dim)
    a2 = jnp.stack([a[0, :out_dim], a[0, out_dim:]], axis=1)

    bn = 1000
    grid = n // bn

    wh, s12 = pl.pallas_call(
        _wh_body,
        grid=(grid,),
        in_specs=[
            pl.BlockSpec((bn, in_dim), lambda i: (i, 0)),
            pl.BlockSpec((in_dim, out_dim), lambda i: (0, 0)),
            pl.BlockSpec((1, out_dim), lambda i: (0, 0)),
            pl.BlockSpec((out_dim, 2), lambda i: (0, 0)),
        ],
        out_specs=[
            pl.BlockSpec((bn, out_dim), lambda i: (i, 0)),
            pl.BlockSpec((bn, 2), lambda i: (i, 0)),
        ],
        out_shape=[
            jax.ShapeDtypeStruct((n, out_dim), jnp.float32),
            jax.ShapeDtypeStruct((n, 2), jnp.float32),
        ],
    )(x, wt, bias, a2)

    zrow = jnp.zeros((n, out_dim), jnp.float32)
    zden = jnp.zeros((n,), jnp.float32)

    tail = n_edges - (n_edges // _CH) * _CH

    mesh = plsc.VectorSubcoreMesh(core_axis_name="c", subcore_axis_name="s")
    sc_fn = pl.kernel(
        functools.partial(_sc_edge_kernel, n, n_edges, out_dim),
        out_type=[
            jax.ShapeDtypeStruct((2, n, out_dim), jnp.float32),
            jax.ShapeDtypeStruct((2, n), jnp.float32),
        ],
        mesh=mesh,
        compiler_params=pltpu.CompilerParams(needs_layout_passes=False),
        scratch_types=[
            pltpu.VMEM((3, _CH), jnp.float32),
            pltpu.VMEM((3, _CH), jnp.float32),
            pltpu.VMEM((3, _CH), jnp.int32),
            pltpu.VMEM((3, _CH), jnp.int32),
            pltpu.VMEM((3, _CH), jnp.int32),
            pltpu.VMEM((3, _CH), jnp.float32),
            pltpu.VMEM((3, _CH, out_dim), jnp.float32),
            pltpu.VMEM((max(tail, 1),), jnp.int32),
            pltpu.VMEM((max(tail, 1),), jnp.int32),
            pltpu.VMEM((max(tail, 1),), jnp.float32),
            pltpu.VMEM((max(tail, 1),), jnp.float32),
            pltpu.VMEM((max(tail, 1),), jnp.float32),
            pltpu.VMEM((1, max(tail, 1), out_dim), jnp.float32),
            pltpu.VMEM_SHARED((n, out_dim), jnp.float32),
            pltpu.VMEM_SHARED((n,), jnp.float32),
            pltpu.SemaphoreType.DMA,
            pltpu.SemaphoreType.DMA,
            pltpu.SemaphoreType.DMA,
            pltpu.SemaphoreType.DMA,
            pltpu.SemaphoreType.DMA,
            pltpu.SemaphoreType.DMA,
            pltpu.SemaphoreType.DMA,
            pltpu.SemaphoreType.DMA,
            pltpu.SemaphoreType.DMA,
            pltpu.SemaphoreType.DMA,
        ],
    )
    agg_part, den_part = sc_fn(edge_index.reshape(2 * n_edges),
                               s12[:, 0], s12[:, 1], wh, zrow, zden)

    den3 = den_part.reshape(2, n, 1)
    out = pl.pallas_call(
        _fin_body,
        grid=(grid,),
        in_specs=[
            pl.BlockSpec((bn, out_dim), lambda i: (i, 0)),
            pl.BlockSpec((1, bn, out_dim), lambda i: (0, i, 0)),
            pl.BlockSpec((1, bn, out_dim), lambda i: (1, i, 0)),
            pl.BlockSpec((1, bn, 1), lambda i: (0, i, 0)),
            pl.BlockSpec((1, bn, 1), lambda i: (1, i, 0)),
        ],
        out_specs=pl.BlockSpec((bn, out_dim), lambda i: (i, 0)),
        out_shape=jax.ShapeDtypeStruct((n, out_dim), jnp.float32),
    )(wh, agg_part, agg_part, den3, den3)

    return out
